# baseline (speedup 1.0000x reference)
import functools

import jax
import jax.numpy as jnp
from jax import lax
from jax.experimental import pallas as pl
from jax.experimental.pallas import tpu as pltpu
from jax.experimental.pallas import tpu_sc as plsc

B = 16384
H = 64
L = 16
NC = 2
NS = 16
NW = NC * NS
BPW = B // NW
CB = 128
NCHUNK = BPW // CB
NBUF = 2
NGROUP = CB // L
KH = H // L


def _body(src_h, rel_h, dst_h, er_h, ei_h, rr_h, ri_h, out_h,
          s_idx, r_idx, d_idx,
          sr_b, si_b, dr_b, di_b, qr_b, qi_b,
          p_v, out_v, sem0, sem1):
    sems = (sem0, sem1)
    wid = lax.axis_index("s") * NC + lax.axis_index("c")
    base = wid * BPW

    stage = []
    for c in range(NCHUNK):
        off = base + c * CB
        stage.append(pltpu.async_copy(src_h.at[pl.ds(off, CB)], s_idx.at[c], sem0))
        stage.append(pltpu.async_copy(rel_h.at[pl.ds(off, CB)], r_idx.at[c], sem0))
        stage.append(pltpu.async_copy(dst_h.at[pl.ds(off, CB)], d_idx.at[c], sem0))
    for cp in stage:
        cp.wait()

    gathers = ((er_h, s_idx, sr_b), (ei_h, s_idx, si_b),
               (er_h, d_idx, dr_b), (ei_h, d_idx, di_b),
               (rr_h, r_idx, qr_b), (ri_h, r_idx, qi_b))

    def issue(c):
        s = c % NBUF
        return [pltpu.async_copy(tab.at[idx.at[c]], buf.at[s], sems[s])
                for tab, idx, buf in gathers]

    def compute(c):
        s = c % NBUF

        def g_body(g, _):
            def e_body(el, __):
                e = g * L + el
                acc = jnp.zeros((L,), jnp.float32)
                for k in range(KH):
                    ds = pl.ds(k * L, L)
                    a = sr_b[s, e, ds]
                    bi = si_b[s, e, ds]
                    cr = dr_b[s, e, ds]
                    ci = di_b[s, e, ds]
                    rr = qr_b[s, e, ds]
                    ri = qi_b[s, e, ds]
                    acc = acc + rr * (a * cr + bi * ci) + ri * (a * ci - bi * cr)
                p_v[pl.ds(el * (L + 1), L)] = acc
                return 0

            lax.fori_loop(0, L, e_body, 0)
            rows = lax.iota(jnp.int32, L) * (L + 1)
            tot = jnp.zeros((L,), jnp.float32)
            for j in range(L):
                tot = tot + plsc.load_gather(p_v, [rows + j])
            out_v[pl.ds(c * CB + g * L, L)] = tot
            return 0

        lax.fori_loop(0, NGROUP, g_body, 0)

    pending = {0: issue(0)}
    for c in range(NCHUNK):
        if c + 1 < NCHUNK:
            pending[c + 1] = issue(c + 1)
        for cp in pending.pop(c):
            cp.wait()
        compute(c)

    pltpu.sync_copy(out_v, out_h.at[pl.ds(base, BPW)])


_sc_call = functools.partial(
    pl.kernel,
    out_type=jax.ShapeDtypeStruct((B,), jnp.float32),
    mesh=plsc.VectorSubcoreMesh(core_axis_name="c", subcore_axis_name="s"),
    compiler_params=pltpu.CompilerParams(
        needs_layout_passes=False, use_tc_tiling_on_sc=False),
    scratch_types=[
        pltpu.VMEM((NCHUNK, CB), jnp.int32),
        pltpu.VMEM((NCHUNK, CB), jnp.int32),
        pltpu.VMEM((NCHUNK, CB), jnp.int32),
        pltpu.VMEM((NBUF, CB, H), jnp.float32),
        pltpu.VMEM((NBUF, CB, H), jnp.float32),
        pltpu.VMEM((NBUF, CB, H), jnp.float32),
        pltpu.VMEM((NBUF, CB, H), jnp.float32),
        pltpu.VMEM((NBUF, CB, H), jnp.float32),
        pltpu.VMEM((NBUF, CB, H), jnp.float32),
        pltpu.VMEM((L * (L + 1),), jnp.float32),
        pltpu.VMEM((BPW,), jnp.float32),
        pltpu.SemaphoreType.DMA,
        pltpu.SemaphoreType.DMA,
    ],
)(_body)


@jax.jit
def kernel(src, rel, dst, ent_real, ent_imag, rel_real, rel_imag):
    return _sc_call(src.astype(jnp.int32), rel.astype(jnp.int32),
                    dst.astype(jnp.int32), ent_real, ent_imag,
                    rel_real, rel_imag)
